# Initial kernel scaffold; baseline (speedup 1.0000x reference)
import jax
import jax.numpy as jnp
from jax.experimental import pallas as pl


def kernel(seq_pads, seq_lens, Wx_q, Wh_q, b_q, Wx_r, Wh_r, b_r, Wx_s, Wh_s, b_s):
    raise NotImplementedError("write your pallas kernel here")


# speedup vs baseline: 11.2011x; 11.2011x over previous
import jax
import jax.numpy as jnp
import numpy as np
from jax.experimental import pallas as pl
from jax.experimental.pallas import tpu as pltpu

_B, _L, _HID = 16, 512, 256
_T_END = 10000.0
_M = 5
_SCALES = (10.0 ** np.arange(_M)).astype(np.float32)
_LN_SCALES = np.log(_SCALES).astype(np.float32)
_C = 32
_NCHUNK = _L // _C


def _scan_kernel(pads_ref, sl_ref, wxq_ref, whq_ref, bq_ref, wxr_ref, whr_ref,
                 br_ref, wxs_ref, whs_ref, bs_ref,
                 bef_ref, aft_ref, delta_ref,
                 hh_ref, dsc_ref, buf_bef, buf_aft,
                 sem_bef, sem_aft):
    g = pl.program_id(0)
    p = jax.lax.rem(g, 2)

    @pl.when(g == 0)
    def _prologue():
        pads = pads_ref[...]
        sl = sl_ref[:, 0:1].astype(jnp.int32)
        cols = jax.lax.broadcasted_iota(jnp.int32, (_B, _L + 1), 1)
        colsL = jax.lax.broadcasted_iota(jnp.int32, (_B, _L), 1)
        diffs = jnp.concatenate(
            [pads[:, 0:1], pads[:, 1:] - pads[:, :-1]], axis=1)
        diffs_ext = jnp.concatenate(
            [diffs, jnp.full((_B, 1), -1.0, jnp.float32)], axis=1)
        t_last = jnp.sum(
            jnp.where(colsL == sl - 1, pads, 0.0), axis=1, keepdims=True)
        delta = jnp.where(cols < sl, diffs_ext, -1.0)
        delta = jnp.where(cols == sl, _T_END - t_last, delta)
        delta_ref[...] = delta
        dshift = jnp.concatenate([delta[:, 0:1], delta[:, :_L]], axis=1)
        for j in range(_NCHUNK):
            dsc_ref[j] = dshift[:, j * _C:(j + 1) * _C]
        dsc_ref[_NCHUNK] = jnp.broadcast_to(dshift[:, _L:_L + 1], (_B, _C))
        hh_ref[...] = jnp.zeros((_M * _B, _HID), jnp.float32)

    @pl.when(g >= 2)
    def _wait_slot():
        pltpu.make_async_copy(
            buf_bef.at[p], bef_ref.at[:, :, pl.ds(0, _C), :], sem_bef.at[p]
        ).wait()
        pltpu.make_async_copy(
            buf_aft.at[p], aft_ref.at[:, :, pl.ds(0, _C), :], sem_aft.at[p]
        ).wait()

    whq = whq_ref[...]
    whr = whr_ref[...]
    whs = whs_ref[...]
    wxq = wxq_ref[0:1, :]
    wxr = wxr_ref[0:1, :]
    wxs = wxs_ref[0:1, :]
    bq = bq_ref[0:1, :]
    br = br_ref[0:1, :]
    bs = bs_ref[0:1, :]

    dt_chunk = dsc_ref[g]
    kiota = jax.lax.broadcasted_iota(jnp.int32, (_B, _C), 1)

    def step(k, carry):
        i = g * _C + k
        dt = jnp.sum(jnp.where(kiota == k, dt_chunk, 0.0),
                     axis=1, keepdims=True)
        valid = dt >= 0.0
        dtc = jnp.where(valid, dt, 0.0)
        hh = hh_ref[...]
        dec80 = jnp.concatenate(
            [jnp.exp(dtc * (-1.0 / _SCALES[m])) for m in range(_M)], axis=0)
        dec = hh * dec80
        dm = [dec[m * _B:(m + 1) * _B] for m in range(_M)]
        h = dm[0] + dm[1] + dm[2] + dm[3] + dm[4]

        ltr = dtc * wxr + jnp.dot(
            h, whr, preferred_element_type=jnp.float32) + br
        a = [-(ltr - _LN_SCALES[m]) ** 2 for m in range(_M)]
        mx = jnp.maximum(jnp.maximum(jnp.maximum(a[0], a[1]),
                                     jnp.maximum(a[2], a[3])), a[4])
        e = [jnp.exp(a[m] - mx) for m in range(_M)]
        rinv = 1.0 / (e[0] + e[1] + e[2] + e[3] + e[4])
        h_ret = (e[0] * dm[0] + e[1] * dm[1] + e[2] * dm[2]
                 + e[3] * dm[3] + e[4] * dm[4]) * rinv

        q = jnp.tanh(dtc * wxq + jnp.dot(
            h_ret, whq, preferred_element_type=jnp.float32) + bq)

        lts = dtc * wxs + jnp.dot(
            h, whs, preferred_element_type=jnp.float32) + bs
        a2 = [-(lts - _LN_SCALES[m]) ** 2 for m in range(_M)]
        mx2 = jnp.maximum(jnp.maximum(jnp.maximum(a2[0], a2[1]),
                                      jnp.maximum(a2[2], a2[3])), a2[4])
        e2 = [jnp.exp(a2[m] - mx2) for m in range(_M)]
        r2inv = 1.0 / (e2[0] + e2[1] + e2[2] + e2[3] + e2[4])
        sw = [e2[m] * r2inv for m in range(_M)]

        new = jnp.concatenate(
            [jnp.where(valid, dm[m] + sw[m] * (q - dm[m]), hh[m * _B:(m + 1) * _B])
             for m in range(_M)], axis=0)
        keep = i >= 1
        hh_ref[...] = jnp.where(keep, new, hh)
        buf_bef[p, :, :, pl.ds(k, 1), :] = jnp.where(
            keep, dec, 0.0).reshape(_M, _B, 1, _HID)
        buf_aft[p, :, :, pl.ds(k, 1), :] = jnp.where(
            keep, new, 0.0).reshape(_M, _B, 1, _HID)
        return carry

    n = jnp.where(g == _NCHUNK, 1, _C)
    jax.lax.fori_loop(0, n, step, 0)

    @pl.when(g < _NCHUNK)
    def _start_full():
        off = g * _C
        pltpu.make_async_copy(
            buf_bef.at[p], bef_ref.at[:, :, pl.ds(off, _C), :], sem_bef.at[p]
        ).start()
        pltpu.make_async_copy(
            buf_aft.at[p], aft_ref.at[:, :, pl.ds(off, _C), :], sem_aft.at[p]
        ).start()

    @pl.when(g == _NCHUNK)
    def _tail():
        cb = pltpu.make_async_copy(
            buf_bef.at[p, :, :, pl.ds(0, 1), :],
            bef_ref.at[:, :, pl.ds(_L, 1), :], sem_bef.at[p])
        ca = pltpu.make_async_copy(
            buf_aft.at[p, :, :, pl.ds(0, 1), :],
            aft_ref.at[:, :, pl.ds(_L, 1), :], sem_aft.at[p])
        cb.start()
        ca.start()
        cb.wait()
        ca.wait()
        q = 1 - p
        pltpu.make_async_copy(
            buf_bef.at[q], bef_ref.at[:, :, pl.ds(0, _C), :], sem_bef.at[q]
        ).wait()
        pltpu.make_async_copy(
            buf_aft.at[q], aft_ref.at[:, :, pl.ds(0, _C), :], sem_aft.at[q]
        ).wait()


def kernel(seq_pads, seq_lens, Wx_q, Wh_q, b_q, Wx_r, Wh_r, b_r, Wx_s, Wh_s, b_s):
    pads = seq_pads[:, :, 0]
    sl2 = jnp.broadcast_to(
        seq_lens.astype(jnp.float32)[:, None], (_B, 128))
    wide = lambda v: jnp.broadcast_to(v.reshape(1, _HID), (8, _HID))

    const_spec2 = lambda shape: pl.BlockSpec(shape, lambda g: (0, 0))

    bef, aft, delta2d = pl.pallas_call(
        _scan_kernel,
        grid=(_NCHUNK + 1,),
        in_specs=[
            const_spec2((_B, _L)),
            const_spec2((_B, 128)),
            const_spec2((8, _HID)),
            const_spec2((_HID, _HID)),
            const_spec2((8, _HID)),
            const_spec2((8, _HID)),
            const_spec2((_HID, _HID)),
            const_spec2((8, _HID)),
            const_spec2((8, _HID)),
            const_spec2((_HID, _HID)),
            const_spec2((8, _HID)),
        ],
        out_specs=[
            pl.BlockSpec(memory_space=pl.ANY),
            pl.BlockSpec(memory_space=pl.ANY),
            const_spec2((_B, _L + 1)),
        ],
        out_shape=[
            jax.ShapeDtypeStruct((_M, _B, _L + 1, _HID), jnp.float32),
            jax.ShapeDtypeStruct((_M, _B, _L + 1, _HID), jnp.float32),
            jax.ShapeDtypeStruct((_B, _L + 1), jnp.float32),
        ],
        scratch_shapes=[
            pltpu.VMEM((_M * _B, _HID), jnp.float32),
            pltpu.VMEM((_NCHUNK + 1, _B, _C), jnp.float32),
            pltpu.VMEM((2, _M, _B, _C, _HID), jnp.float32),
            pltpu.VMEM((2, _M, _B, _C, _HID), jnp.float32),
            pltpu.SemaphoreType.DMA((2,)),
            pltpu.SemaphoreType.DMA((2,)),
        ],
        compiler_params=pltpu.CompilerParams(
            dimension_semantics=("arbitrary",)),
    )(pads, sl2, wide(Wx_q), Wh_q, wide(b_q), wide(Wx_r), Wh_r, wide(b_r),
      wide(Wx_s), Wh_s, wide(b_s))

    return bef, aft, delta2d[:, :, None]


# speedup vs baseline: 15.2190x; 1.3587x over previous
import jax
import jax.numpy as jnp
import numpy as np
from jax.experimental import pallas as pl
from jax.experimental.pallas import tpu as pltpu

_B, _L, _HID = 16, 512, 256
_T_END = 10000.0
_M = 5
_SCALES = (10.0 ** np.arange(_M)).astype(np.float32)
_LN_SCALES = np.log(_SCALES).astype(np.float32)
_C = 32
_NCHUNK = _L // _C


def _scan_kernel(pads_ref, sl_ref, wxq_ref, whq_ref, bq_ref, wxrs_ref,
                 whrs_ref, brs_ref,
                 bef_ref, aft_ref, delta_ref,
                 hh_ref, dsc_ref, buf_bef, buf_aft,
                 sem_bef, sem_aft):
    g = pl.program_id(0)
    p = jax.lax.rem(g, 2)

    @pl.when(g == 0)
    def _prologue():
        pads = pads_ref[...]
        sl = sl_ref[:, 0:1].astype(jnp.int32)
        cols = jax.lax.broadcasted_iota(jnp.int32, (_B, _L + 1), 1)
        colsL = jax.lax.broadcasted_iota(jnp.int32, (_B, _L), 1)
        diffs = jnp.concatenate(
            [pads[:, 0:1], pads[:, 1:] - pads[:, :-1]], axis=1)
        diffs_ext = jnp.concatenate(
            [diffs, jnp.full((_B, 1), -1.0, jnp.float32)], axis=1)
        t_last = jnp.sum(
            jnp.where(colsL == sl - 1, pads, 0.0), axis=1, keepdims=True)
        delta = jnp.where(cols < sl, diffs_ext, -1.0)
        delta = jnp.where(cols == sl, _T_END - t_last, delta)
        delta_ref[...] = delta
        dshift = jnp.concatenate([delta[:, 0:1], delta[:, :_L]], axis=1)
        for j in range(_NCHUNK):
            dsc_ref[j] = dshift[:, j * _C:(j + 1) * _C]
        dsc_ref[_NCHUNK] = jnp.broadcast_to(dshift[:, _L:_L + 1], (_B, _C))
        hh_ref[...] = jnp.zeros((_M * _B, _HID), jnp.float32)

    @pl.when(g >= 2)
    def _wait_slot():
        pltpu.make_async_copy(
            buf_bef.at[p], bef_ref.at[:, :, pl.ds(0, _C), :], sem_bef.at[p]
        ).wait()
        pltpu.make_async_copy(
            buf_aft.at[p], aft_ref.at[:, :, pl.ds(0, _C), :], sem_aft.at[p]
        ).wait()

    whq = whq_ref[...]
    whrs = whrs_ref[...]
    wxq = wxq_ref[0:1, :]
    wxrs = wxrs_ref[0:1, :]
    bq = bq_ref[0:1, :]
    brs = brs_ref[0:1, :]

    dt_chunk = dsc_ref[g]
    valid_chunk = dt_chunk >= 0.0
    dtc_chunk = jnp.where(valid_chunk, dt_chunk, 0.0)
    decay_chunk = [jnp.exp(dtc_chunk * (-1.0 / _SCALES[m])) for m in range(_M)]

    hh = hh_ref[...]
    hm = [hh[m * _B:(m + 1) * _B] for m in range(_M)]

    for k in range(_C):
        valid = valid_chunk[:, k:k + 1]
        dtc = dtc_chunk[:, k:k + 1]
        dm = [hm[m] * decay_chunk[m][:, k:k + 1] for m in range(_M)]
        h = dm[0] + dm[1] + dm[2] + dm[3] + dm[4]

        lrs = dtc * wxrs + jnp.dot(
            h, whrs, preferred_element_type=jnp.float32) + brs
        ltr = lrs[:, :_HID]
        lts = lrs[:, _HID:]

        a = [-(ltr - _LN_SCALES[m]) ** 2 for m in range(_M)]
        mx = jnp.maximum(jnp.maximum(jnp.maximum(a[0], a[1]),
                                     jnp.maximum(a[2], a[3])), a[4])
        e = [jnp.exp(a[m] - mx) for m in range(_M)]
        rinv = 1.0 / (e[0] + e[1] + e[2] + e[3] + e[4])
        h_ret = (e[0] * dm[0] + e[1] * dm[1] + e[2] * dm[2]
                 + e[3] * dm[3] + e[4] * dm[4]) * rinv

        q = jnp.tanh(dtc * wxq + jnp.dot(
            h_ret, whq, preferred_element_type=jnp.float32) + bq)

        a2 = [-(lts - _LN_SCALES[m]) ** 2 for m in range(_M)]
        mx2 = jnp.maximum(jnp.maximum(jnp.maximum(a2[0], a2[1]),
                                      jnp.maximum(a2[2], a2[3])), a2[4])
        e2 = [jnp.exp(a2[m] - mx2) for m in range(_M)]
        r2inv = 1.0 / (e2[0] + e2[1] + e2[2] + e2[3] + e2[4])

        i_first = (g == 0) & (k == 0)
        i_dead = (g == _NCHUNK) & (k >= 1)
        upd = valid & jnp.logical_not(jnp.logical_or(i_first, i_dead))
        hm = [jnp.where(upd, dm[m] + (e2[m] * r2inv) * (q - dm[m]), hm[m])
              for m in range(_M)]
        for m in range(_M):
            buf_bef[p, m, :, k:k + 1, :] = dm[m].reshape(_B, 1, _HID)
            buf_aft[p, m, :, k:k + 1, :] = hm[m].reshape(_B, 1, _HID)

    hh_ref[...] = jnp.concatenate(hm, axis=0)

    @pl.when(g < _NCHUNK)
    def _start_full():
        off = g * _C
        pltpu.make_async_copy(
            buf_bef.at[p], bef_ref.at[:, :, pl.ds(off, _C), :], sem_bef.at[p]
        ).start()
        pltpu.make_async_copy(
            buf_aft.at[p], aft_ref.at[:, :, pl.ds(off, _C), :], sem_aft.at[p]
        ).start()

    @pl.when(g == _NCHUNK)
    def _tail():
        cb = pltpu.make_async_copy(
            buf_bef.at[p, :, :, pl.ds(0, 1), :],
            bef_ref.at[:, :, pl.ds(_L, 1), :], sem_bef.at[p])
        ca = pltpu.make_async_copy(
            buf_aft.at[p, :, :, pl.ds(0, 1), :],
            aft_ref.at[:, :, pl.ds(_L, 1), :], sem_aft.at[p])
        cb.start()
        ca.start()
        cb.wait()
        ca.wait()
        q2 = 1 - p
        pltpu.make_async_copy(
            buf_bef.at[q2], bef_ref.at[:, :, pl.ds(0, _C), :], sem_bef.at[q2]
        ).wait()
        pltpu.make_async_copy(
            buf_aft.at[q2], aft_ref.at[:, :, pl.ds(0, _C), :], sem_aft.at[q2]
        ).wait()


def kernel(seq_pads, seq_lens, Wx_q, Wh_q, b_q, Wx_r, Wh_r, b_r, Wx_s, Wh_s, b_s):
    pads = seq_pads[:, :, 0]
    sl2 = jnp.broadcast_to(
        seq_lens.astype(jnp.float32)[:, None], (_B, 128))
    wide = lambda v, w: jnp.broadcast_to(v.reshape(1, w), (8, w))
    wxrs = jnp.concatenate([Wx_r, Wx_s], axis=1)
    whrs = jnp.concatenate([Wh_r, Wh_s], axis=1)
    brs = jnp.concatenate([b_r, b_s], axis=0)

    const_spec2 = lambda shape: pl.BlockSpec(shape, lambda g: (0, 0))

    bef, aft, delta2d = pl.pallas_call(
        _scan_kernel,
        grid=(_NCHUNK + 1,),
        in_specs=[
            const_spec2((_B, _L)),
            const_spec2((_B, 128)),
            const_spec2((8, _HID)),
            const_spec2((_HID, _HID)),
            const_spec2((8, _HID)),
            const_spec2((8, 2 * _HID)),
            const_spec2((_HID, 2 * _HID)),
            const_spec2((8, 2 * _HID)),
        ],
        out_specs=[
            pl.BlockSpec(memory_space=pl.ANY),
            pl.BlockSpec(memory_space=pl.ANY),
            const_spec2((_B, _L + 1)),
        ],
        out_shape=[
            jax.ShapeDtypeStruct((_M, _B, _L + 1, _HID), jnp.float32),
            jax.ShapeDtypeStruct((_M, _B, _L + 1, _HID), jnp.float32),
            jax.ShapeDtypeStruct((_B, _L + 1), jnp.float32),
        ],
        scratch_shapes=[
            pltpu.VMEM((_M * _B, _HID), jnp.float32),
            pltpu.VMEM((_NCHUNK + 1, _B, _C), jnp.float32),
            pltpu.VMEM((2, _M, _B, _C, _HID), jnp.float32),
            pltpu.VMEM((2, _M, _B, _C, _HID), jnp.float32),
            pltpu.SemaphoreType.DMA((2,)),
            pltpu.SemaphoreType.DMA((2,)),
        ],
        compiler_params=pltpu.CompilerParams(
            dimension_semantics=("arbitrary",)),
    )(pads, sl2, wide(Wx_q, _HID), Wh_q, wide(b_q, _HID),
      wide(wxrs, 2 * _HID), whrs, wide(brs, 2 * _HID))

    return bef, aft, delta2d[:, :, None]


# speedup vs baseline: 16.4575x; 1.0814x over previous
import jax
import jax.numpy as jnp
import numpy as np
from jax.experimental import pallas as pl
from jax.experimental.pallas import tpu as pltpu

_B, _L, _HID = 16, 512, 256
_T_END = 10000.0
_M = 5
_SCALES = (10.0 ** np.arange(_M)).astype(np.float32)
_LN_SCALES = np.log(_SCALES).astype(np.float32)
_C = 32
_NCHUNK = _L // _C


def _scan_kernel(pads_ref, sl_ref, wxq_ref, whq_ref, bq_ref, wxrs_ref,
                 whrs_ref, brs_ref,
                 bef_ref, aft_ref, delta_ref,
                 hh_ref, dsc_ref, buf_bef, buf_aft,
                 sem_bef, sem_aft):
    g = pl.program_id(0)
    p = jax.lax.rem(g, 2)

    @pl.when(g == 0)
    def _prologue():
        pads = pads_ref[...]
        sl = sl_ref[:, 0:1].astype(jnp.int32)
        cols = jax.lax.broadcasted_iota(jnp.int32, (_B, _L + 1), 1)
        colsL = jax.lax.broadcasted_iota(jnp.int32, (_B, _L), 1)
        diffs = jnp.concatenate(
            [pads[:, 0:1], pads[:, 1:] - pads[:, :-1]], axis=1)
        diffs_ext = jnp.concatenate(
            [diffs, jnp.full((_B, 1), -1.0, jnp.float32)], axis=1)
        t_last = jnp.sum(
            jnp.where(colsL == sl - 1, pads, 0.0), axis=1, keepdims=True)
        delta = jnp.where(cols < sl, diffs_ext, -1.0)
        delta = jnp.where(cols == sl, _T_END - t_last, delta)
        delta_ref[...] = delta
        dshift = jnp.concatenate([delta[:, 0:1], delta[:, :_L]], axis=1)
        for j in range(_NCHUNK):
            dsc_ref[j] = dshift[:, j * _C:(j + 1) * _C]
        dsc_ref[_NCHUNK] = jnp.broadcast_to(dshift[:, _L:_L + 1], (_B, _C))
        hh_ref[...] = jnp.zeros((_M * _B, _HID), jnp.float32)

    @pl.when(g >= 2)
    def _wait_slot():
        pltpu.make_async_copy(
            buf_bef.at[p], bef_ref.at[:, :, pl.ds(0, _C), :], sem_bef.at[p]
        ).wait()
        pltpu.make_async_copy(
            buf_aft.at[p], aft_ref.at[:, :, pl.ds(0, _C), :], sem_aft.at[p]
        ).wait()

    whq = whq_ref[...]
    whrs = whrs_ref[...]
    wxq = wxq_ref[0:1, :]
    wxrs = wxrs_ref[0:1, :]
    bq = bq_ref[0:1, :]
    brs = brs_ref[0:1, :]

    dt_chunk = dsc_ref[g]
    valid_chunk = dt_chunk >= 0.0
    dtc_chunk = jnp.where(valid_chunk, dt_chunk, 0.0)
    decay_chunk = [jnp.exp(dtc_chunk * (-1.0 / _SCALES[m])) for m in range(_M)]

    hh = hh_ref[...]
    _HB = _B // 2
    hm = [[hh[m * _B + hb * _HB:m * _B + (hb + 1) * _HB] for m in range(_M)]
          for hb in range(2)]

    def one_step(k, hb, hmh):
        r0 = hb * _HB
        valid = valid_chunk[r0:r0 + _HB, k:k + 1]
        dtc = dtc_chunk[r0:r0 + _HB, k:k + 1]
        dm = [hmh[m] * decay_chunk[m][r0:r0 + _HB, k:k + 1] for m in range(_M)]
        h = dm[0] + dm[1] + dm[2] + dm[3] + dm[4]

        lrs = dtc * wxrs + jnp.dot(
            h, whrs, preferred_element_type=jnp.float32) + brs
        ltr = lrs[:, :_HID]
        lts = lrs[:, _HID:]

        a = [-(ltr - _LN_SCALES[m]) ** 2 for m in range(_M)]
        mx = jnp.maximum(jnp.maximum(jnp.maximum(a[0], a[1]),
                                     jnp.maximum(a[2], a[3])), a[4])
        e = [jnp.exp(a[m] - mx) for m in range(_M)]
        rinv = 1.0 / (e[0] + e[1] + e[2] + e[3] + e[4])
        h_ret = (e[0] * dm[0] + e[1] * dm[1] + e[2] * dm[2]
                 + e[3] * dm[3] + e[4] * dm[4]) * rinv

        q = jnp.tanh(dtc * wxq + jnp.dot(
            h_ret, whq, preferred_element_type=jnp.float32) + bq)

        a2 = [-(lts - _LN_SCALES[m]) ** 2 for m in range(_M)]
        mx2 = jnp.maximum(jnp.maximum(jnp.maximum(a2[0], a2[1]),
                                      jnp.maximum(a2[2], a2[3])), a2[4])
        e2 = [jnp.exp(a2[m] - mx2) for m in range(_M)]
        r2inv = 1.0 / (e2[0] + e2[1] + e2[2] + e2[3] + e2[4])

        i_first = (g == 0) & (k == 0)
        i_dead = (g == _NCHUNK) & (k >= 1)
        upd = valid & jnp.logical_not(jnp.logical_or(i_first, i_dead))
        hmh = [jnp.where(upd, dm[m] + (e2[m] * r2inv) * (q - dm[m]), hmh[m])
               for m in range(_M)]
        for m in range(_M):
            buf_bef[p, m, r0:r0 + _HB, k:k + 1, :] = dm[m].reshape(
                _HB, 1, _HID)
            buf_aft[p, m, r0:r0 + _HB, k:k + 1, :] = hmh[m].reshape(
                _HB, 1, _HID)
        return hmh

    for k in range(_C):
        hm = [one_step(k, hb, hm[hb]) for hb in range(2)]

    hh_ref[...] = jnp.concatenate(
        [hm[hb][m] for m in range(_M) for hb in range(2)], axis=0)

    @pl.when(g < _NCHUNK)
    def _start_full():
        off = g * _C
        pltpu.make_async_copy(
            buf_bef.at[p], bef_ref.at[:, :, pl.ds(off, _C), :], sem_bef.at[p]
        ).start()
        pltpu.make_async_copy(
            buf_aft.at[p], aft_ref.at[:, :, pl.ds(off, _C), :], sem_aft.at[p]
        ).start()

    @pl.when(g == _NCHUNK)
    def _tail():
        cb = pltpu.make_async_copy(
            buf_bef.at[p, :, :, pl.ds(0, 1), :],
            bef_ref.at[:, :, pl.ds(_L, 1), :], sem_bef.at[p])
        ca = pltpu.make_async_copy(
            buf_aft.at[p, :, :, pl.ds(0, 1), :],
            aft_ref.at[:, :, pl.ds(_L, 1), :], sem_aft.at[p])
        cb.start()
        ca.start()
        cb.wait()
        ca.wait()
        q2 = 1 - p
        pltpu.make_async_copy(
            buf_bef.at[q2], bef_ref.at[:, :, pl.ds(0, _C), :], sem_bef.at[q2]
        ).wait()
        pltpu.make_async_copy(
            buf_aft.at[q2], aft_ref.at[:, :, pl.ds(0, _C), :], sem_aft.at[q2]
        ).wait()


def kernel(seq_pads, seq_lens, Wx_q, Wh_q, b_q, Wx_r, Wh_r, b_r, Wx_s, Wh_s, b_s):
    pads = seq_pads[:, :, 0]
    sl2 = jnp.broadcast_to(
        seq_lens.astype(jnp.float32)[:, None], (_B, 128))
    wide = lambda v, w: jnp.broadcast_to(v.reshape(1, w), (8, w))
    wxrs = jnp.concatenate([Wx_r, Wx_s], axis=1)
    whrs = jnp.concatenate([Wh_r, Wh_s], axis=1)
    brs = jnp.concatenate([b_r, b_s], axis=0)

    const_spec2 = lambda shape: pl.BlockSpec(shape, lambda g: (0, 0))

    bef, aft, delta2d = pl.pallas_call(
        _scan_kernel,
        grid=(_NCHUNK + 1,),
        in_specs=[
            const_spec2((_B, _L)),
            const_spec2((_B, 128)),
            const_spec2((8, _HID)),
            const_spec2((_HID, _HID)),
            const_spec2((8, _HID)),
            const_spec2((8, 2 * _HID)),
            const_spec2((_HID, 2 * _HID)),
            const_spec2((8, 2 * _HID)),
        ],
        out_specs=[
            pl.BlockSpec(memory_space=pl.ANY),
            pl.BlockSpec(memory_space=pl.ANY),
            const_spec2((_B, _L + 1)),
        ],
        out_shape=[
            jax.ShapeDtypeStruct((_M, _B, _L + 1, _HID), jnp.float32),
            jax.ShapeDtypeStruct((_M, _B, _L + 1, _HID), jnp.float32),
            jax.ShapeDtypeStruct((_B, _L + 1), jnp.float32),
        ],
        scratch_shapes=[
            pltpu.VMEM((_M * _B, _HID), jnp.float32),
            pltpu.VMEM((_NCHUNK + 1, _B, _C), jnp.float32),
            pltpu.VMEM((2, _M, _B, _C, _HID), jnp.float32),
            pltpu.VMEM((2, _M, _B, _C, _HID), jnp.float32),
            pltpu.SemaphoreType.DMA((2,)),
            pltpu.SemaphoreType.DMA((2,)),
        ],
        compiler_params=pltpu.CompilerParams(
            dimension_semantics=("arbitrary",)),
    )(pads, sl2, wide(Wx_q, _HID), Wh_q, wide(b_q, _HID),
      wide(wxrs, 2 * _HID), whrs, wide(brs, 2 * _HID))

    return bef, aft, delta2d[:, :, None]


# speedup vs baseline: 17.0000x; 1.0330x over previous
import jax
import jax.numpy as jnp
import numpy as np
from jax.experimental import pallas as pl
from jax.experimental.pallas import tpu as pltpu

_B, _L, _HID = 16, 512, 256
_T_END = 10000.0
_M = 5
_SCALES = (10.0 ** np.arange(_M)).astype(np.float32)
_LN_SCALES = np.log(_SCALES).astype(np.float32)
_C = 32
_NCHUNK = _L // _C


def _scan_kernel(sl_ref, pads_ref, wxq_ref, whq_ref, bq_ref, wxr_ref, whr_ref,
                 br_ref, wxs_ref, whs_ref, bs_ref,
                 bef_ref, aft_ref, delta_ref,
                 hh_ref, dsc_ref, buf_bef, buf_aft,
                 sem_bef, sem_aft):
    g = pl.program_id(0)
    p = jax.lax.rem(g, 2)

    @pl.when(g == 0)
    def _prologue():
        pads = pads_ref[...]
        cols = jax.lax.broadcasted_iota(jnp.int32, (_B, _L + 1), 1)
        colsL = jax.lax.broadcasted_iota(jnp.int32, (_B, _L), 1)
        diffs = jnp.concatenate(
            [pads[:, 0:1], pads[:, 1:] - pads[:, :-1]], axis=1)
        diffs_ext = jnp.concatenate(
            [diffs, jnp.full((_B, 1), -1.0, jnp.float32)], axis=1)
        rows = jax.lax.broadcasted_iota(jnp.int32, (_B, 1), 0)
        sl = jnp.zeros((_B, 1), jnp.int32)
        for b in range(_B):
            sl = jnp.where(rows == b, sl_ref[b], sl)
        t_last = jnp.sum(
            jnp.where(colsL == sl - 1, pads, 0.0), axis=1, keepdims=True)
        delta = jnp.where(cols < sl, diffs_ext, -1.0)
        delta = jnp.where(cols == sl, _T_END - t_last, delta)
        delta_ref[...] = delta
        dshift = jnp.concatenate([delta[:, 0:1], delta[:, :_L]], axis=1)
        for j in range(_NCHUNK):
            dsc_ref[j] = dshift[:, j * _C:(j + 1) * _C]
        dsc_ref[_NCHUNK] = jnp.broadcast_to(dshift[:, _L:_L + 1], (_B, _C))
        hh_ref[...] = jnp.zeros((_M * _B, _HID), jnp.float32)

    @pl.when(g >= 2)
    def _wait_slot():
        pltpu.make_async_copy(
            buf_bef.at[p], bef_ref.at[:, :, pl.ds(0, _C), :], sem_bef.at[p]
        ).wait()
        pltpu.make_async_copy(
            buf_aft.at[p], aft_ref.at[:, :, pl.ds(0, _C), :], sem_aft.at[p]
        ).wait()

    whq = whq_ref[...]
    whr = whr_ref[...]
    whs = whs_ref[...]
    wxq = wxq_ref[...]
    wxr = wxr_ref[...]
    wxs = wxs_ref[...]
    bq = bq_ref[...]
    br = br_ref[...]
    bs = bs_ref[...]

    dt_chunk = dsc_ref[g]
    valid_chunk = dt_chunk >= 0.0
    dtc_chunk = jnp.where(valid_chunk, dt_chunk, 0.0)
    decay_chunk = [jnp.exp(dtc_chunk * (-1.0 / _SCALES[m])) for m in range(_M)]

    hh = hh_ref[...]
    _HB = _B // 2
    hm = [[hh[m * _B + hb * _HB:m * _B + (hb + 1) * _HB] for m in range(_M)]
          for hb in range(2)]

    def one_step(k, hb, hmh):
        r0 = hb * _HB
        valid = valid_chunk[r0:r0 + _HB, k:k + 1]
        dtc = dtc_chunk[r0:r0 + _HB, k:k + 1]
        dm = [hmh[m] * decay_chunk[m][r0:r0 + _HB, k:k + 1] for m in range(_M)]
        h = dm[0] + dm[1] + dm[2] + dm[3] + dm[4]

        ltr = dtc * wxr + jnp.dot(
            h, whr, preferred_element_type=jnp.float32) + br
        lts = dtc * wxs + jnp.dot(
            h, whs, preferred_element_type=jnp.float32) + bs

        a = [-(ltr - _LN_SCALES[m]) ** 2 for m in range(_M)]
        mx = jnp.maximum(jnp.maximum(jnp.maximum(a[0], a[1]),
                                     jnp.maximum(a[2], a[3])), a[4])
        e = [jnp.exp(a[m] - mx) for m in range(_M)]
        rinv = 1.0 / (e[0] + e[1] + e[2] + e[3] + e[4])
        h_ret = (e[0] * dm[0] + e[1] * dm[1] + e[2] * dm[2]
                 + e[3] * dm[3] + e[4] * dm[4]) * rinv

        q = jnp.tanh(dtc * wxq + jnp.dot(
            h_ret, whq, preferred_element_type=jnp.float32) + bq)

        a2 = [-(lts - _LN_SCALES[m]) ** 2 for m in range(_M)]
        mx2 = jnp.maximum(jnp.maximum(jnp.maximum(a2[0], a2[1]),
                                      jnp.maximum(a2[2], a2[3])), a2[4])
        e2 = [jnp.exp(a2[m] - mx2) for m in range(_M)]
        r2inv = 1.0 / (e2[0] + e2[1] + e2[2] + e2[3] + e2[4])

        i_first = (g == 0) & (k == 0)
        i_dead = (g == _NCHUNK) & (k >= 1)
        upd = valid & jnp.logical_not(jnp.logical_or(i_first, i_dead))
        hmh = [jnp.where(upd, dm[m] + (e2[m] * r2inv) * (q - dm[m]), hmh[m])
               for m in range(_M)]
        for m in range(_M):
            buf_bef[p, m, r0:r0 + _HB, k:k + 1, :] = dm[m].reshape(
                _HB, 1, _HID)
            buf_aft[p, m, r0:r0 + _HB, k:k + 1, :] = hmh[m].reshape(
                _HB, 1, _HID)
        return hmh

    for k in range(_C):
        hm = [one_step(k, hb, hm[hb]) for hb in range(2)]

    hh_ref[...] = jnp.concatenate(
        [hm[hb][m] for m in range(_M) for hb in range(2)], axis=0)

    @pl.when(g < _NCHUNK)
    def _start_full():
        off = g * _C
        pltpu.make_async_copy(
            buf_bef.at[p], bef_ref.at[:, :, pl.ds(off, _C), :], sem_bef.at[p]
        ).start()
        pltpu.make_async_copy(
            buf_aft.at[p], aft_ref.at[:, :, pl.ds(off, _C), :], sem_aft.at[p]
        ).start()

    @pl.when(g == _NCHUNK)
    def _tail():
        cb = pltpu.make_async_copy(
            buf_bef.at[p, :, :, pl.ds(0, 1), :],
            bef_ref.at[:, :, pl.ds(_L, 1), :], sem_bef.at[p])
        ca = pltpu.make_async_copy(
            buf_aft.at[p, :, :, pl.ds(0, 1), :],
            aft_ref.at[:, :, pl.ds(_L, 1), :], sem_aft.at[p])
        cb.start()
        ca.start()
        cb.wait()
        ca.wait()
        q2 = 1 - p
        pltpu.make_async_copy(
            buf_bef.at[q2], bef_ref.at[:, :, pl.ds(0, _C), :], sem_bef.at[q2]
        ).wait()
        pltpu.make_async_copy(
            buf_aft.at[q2], aft_ref.at[:, :, pl.ds(0, _C), :], sem_aft.at[q2]
        ).wait()


def kernel(seq_pads, seq_lens, Wx_q, Wh_q, b_q, Wx_r, Wh_r, b_r, Wx_s, Wh_s, b_s):
    pads = seq_pads.reshape(_B, _L)
    const_spec2 = lambda shape: pl.BlockSpec(shape, lambda g: (0, 0))
    row = lambda v: v.reshape(1, _HID)

    bef, aft, delta2d = pl.pallas_call(
        _scan_kernel,
        grid=(_NCHUNK + 1,),
        in_specs=[
            pl.BlockSpec(memory_space=pltpu.MemorySpace.SMEM),
            const_spec2((_B, _L)),
            const_spec2((1, _HID)),
            const_spec2((_HID, _HID)),
            const_spec2((1, _HID)),
            const_spec2((1, _HID)),
            const_spec2((_HID, _HID)),
            const_spec2((1, _HID)),
            const_spec2((1, _HID)),
            const_spec2((_HID, _HID)),
            const_spec2((1, _HID)),
        ],
        out_specs=[
            pl.BlockSpec(memory_space=pl.ANY),
            pl.BlockSpec(memory_space=pl.ANY),
            const_spec2((_B, _L + 1)),
        ],
        out_shape=[
            jax.ShapeDtypeStruct((_M, _B, _L + 1, _HID), jnp.float32),
            jax.ShapeDtypeStruct((_M, _B, _L + 1, _HID), jnp.float32),
            jax.ShapeDtypeStruct((_B, _L + 1), jnp.float32),
        ],
        scratch_shapes=[
            pltpu.VMEM((_M * _B, _HID), jnp.float32),
            pltpu.VMEM((_NCHUNK + 1, _B, _C), jnp.float32),
            pltpu.VMEM((2, _M, _B, _C, _HID), jnp.float32),
            pltpu.VMEM((2, _M, _B, _C, _HID), jnp.float32),
            pltpu.SemaphoreType.DMA((2,)),
            pltpu.SemaphoreType.DMA((2,)),
        ],
        compiler_params=pltpu.CompilerParams(
            dimension_semantics=("arbitrary",)),
    )(seq_lens, pads, Wx_q, Wh_q, row(b_q), Wx_r, Wh_r, row(b_r),
      Wx_s, Wh_s, row(b_s))

    return bef, aft, delta2d[:, :, None]
